# baseline (speedup 1.0000x reference)
import functools

import jax
import jax.numpy as jnp
from jax import lax
from jax.experimental import pallas as pl
from jax.experimental.pallas import tpu as pltpu
from jax.experimental.pallas import tpu_sc as plsc

B = 16384
EMB_DIM = 32
BERT_DIM = 768
L = 16
NC, NS = 2, 16
NW = NC * NS
BPW = B // NW
EPR = 128 // EMB_DIM
N_EMB_ROWS = 100000 * EMB_DIM // 128

CHT = 32
NCHT = BPW // CHT
CHE = 64
NCHE = BPW // CHE

_GATHER_DNUMS = lax.GatherDimensionNumbers(
    offset_dims=(), collapsed_slice_dims=(0,), start_index_map=(0,))


def _lane_shuffle(v, idx):
    return lax.gather(v, idx[:, None], _GATHER_DNUMS, (1,),
                      mode=lax.GatherScatterMode.PROMISE_IN_BOUNDS)


def _mesh():
    return plsc.VectorSubcoreMesh(core_axis_name="c", subcore_axis_name="s")


def _load_ids(uidf_hbm, iidf_hbm, uidf_v, iidf_v, idq, wid, sem):
    idq[pl.ds(0, L)] = jnp.broadcast_to(wid, (L,)).astype(jnp.int32)
    row = idq.at[pl.ds(0, 1)]
    cu = pltpu.make_async_copy(uidf_hbm.at[row], uidf_v, sem)
    ci = pltpu.make_async_copy(iidf_hbm.at[row], iidf_v, sem)
    cu.start()
    ci.start()
    cu.wait()
    ci.wait()


def _store_i32(dst_ref, ds, value):
    dst_ref[ds] = value


@functools.partial(
    pl.kernel,
    out_type=jax.ShapeDtypeStruct((B,), jnp.float32),
    mesh=_mesh(),
    compiler_params=pltpu.CompilerParams(needs_layout_passes=False),
    scratch_types=[
        pltpu.VMEM((1, BPW), jnp.float32),
        pltpu.VMEM((1, BPW), jnp.float32),
        pltpu.VMEM((L,), jnp.int32),
        pltpu.VMEM((2, CHT), jnp.int32),
        pltpu.VMEM((2, CHT), jnp.int32),
        pltpu.VMEM((2, CHT, BERT_DIM), jnp.float32),
        pltpu.VMEM((2, CHT, BERT_DIM), jnp.float32),
        pltpu.VMEM((BPW,), jnp.float32),
        pltpu.SemaphoreType.DMA((2,)),
    ],
)
def _sc_text(uidf_hbm, iidf_hbm, utext_hbm, itext_hbm, out_hbm,
             uidf_v, iidf_v, idq, uix_c, iix_c, ut_v, it_v, out_v, sem):
    wid = lax.axis_index("s") * NC + lax.axis_index("c")
    base = wid * BPW
    _load_ids(uidf_hbm, iidf_hbm, uidf_v, iidf_v, idq, wid, sem.at[0])
    lane = lax.iota(jnp.int32, L)
    lane0 = lane == 0

    def issue_chunk(j, p):
        for g in range(CHT // L):
            ds = pl.ds(g * L, L)
            uix_c[p, ds] = plsc.bitcast(
                uidf_v[0, pl.ds(j * CHT + g * L, L)], jnp.int32)
            iix_c[p, ds] = plsc.bitcast(
                iidf_v[0, pl.ds(j * CHT + g * L, L)], jnp.int32)
        cps = _chunk_cps(p)
        for c in cps:
            c.start()

    def _chunk_cps(p):
        return [
            pltpu.make_async_copy(utext_hbm.at[uix_c.at[p]], ut_v.at[p],
                                  sem.at[p]),
            pltpu.make_async_copy(itext_hbm.at[iix_c.at[p]], it_v.at[p],
                                  sem.at[p]),
        ]

    issue_chunk(0, 0)

    def chunk_body(j, carry):
        p = lax.rem(j, 2)
        q = 1 - p

        @pl.when(j < NCHT - 1)
        def _issue_next():
            issue_chunk(j + 1, q)

        for c in _chunk_cps(p):
            c.wait()

        def pair_body(i, carry2):
            acc = ut_v[p, i, pl.ds(0, L)] * it_v[p, i, pl.ds(0, L)]
            for t in range(1, BERT_DIM // L):
                acc = acc + (ut_v[p, i, pl.ds(t * L, L)]
                             * it_v[p, i, pl.ds(t * L, L)])
            for sh in (8, 4, 2, 1):
                acc = acc + _lane_shuffle(acc, lane ^ sh)
            pos = jnp.broadcast_to(j * CHT + i, (L,)).astype(jnp.int32)
            plsc.store_scatter(out_v, [pos], acc, mask=lane0)
            return carry2

        lax.fori_loop(0, CHT, pair_body, 0)
        return carry

    lax.fori_loop(0, NCHT, chunk_body, 0)
    pltpu.sync_copy(out_v, out_hbm.at[pl.ds(base, BPW)])


@functools.partial(
    pl.kernel,
    out_type=jax.ShapeDtypeStruct((B,), jnp.float32),
    mesh=_mesh(),
    compiler_params=pltpu.CompilerParams(needs_layout_passes=False),
    scratch_types=[
        pltpu.VMEM((1, BPW), jnp.float32),
        pltpu.VMEM((1, BPW), jnp.float32),
        pltpu.VMEM((L,), jnp.int32),
        pltpu.VMEM((BPW,), jnp.int32),
        pltpu.VMEM((BPW,), jnp.int32),
        pltpu.VMEM((2, EMB_DIM, CHE), jnp.float32),
        pltpu.VMEM((2, EMB_DIM, CHE), jnp.float32),
        pltpu.VMEM((2, CHE), jnp.float32),
        pltpu.VMEM((2, CHE), jnp.float32),
        pltpu.VMEM((BPW,), jnp.float32),
        pltpu.VMEM((L,), jnp.float32),
        pltpu.SemaphoreType.DMA((2,)),
    ],
)
def _sc_emb(*refs):
    (uidf_hbm, iidf_hbm) = refs[0:2]
    ucol_hbm = refs[2:2 + EMB_DIM]
    icol_hbm = refs[2 + EMB_DIM:2 + 2 * EMB_DIM]
    (ubias_hbm, ibias_hbm, bias16_hbm, out_hbm,
     uidf_v, iidf_v, idq, uid_v, iid_v, ue_v, ie_v,
     ub_v, ib_v, out_v, bias_v, sem) = refs[2 + 2 * EMB_DIM:]
    wid = lax.axis_index("s") * NC + lax.axis_index("c")
    base = wid * BPW
    _load_ids(uidf_hbm, iidf_hbm, uidf_v, iidf_v, idq, wid, sem.at[0])
    pltpu.sync_copy(bias16_hbm, bias_v)
    bias_vec = bias_v[pl.ds(0, L)]

    def ids_body(g, carry):
        ds = pl.ds(g * L, L)
        uid_v[ds] = plsc.bitcast(uidf_v[0, ds], jnp.int32)
        iid_v[ds] = plsc.bitcast(iidf_v[0, ds], jnp.int32)
        return carry

    lax.fori_loop(0, BPW // L, ids_body, 0)

    def _chunk_cps(j, p):
        uids = uid_v.at[pl.ds(j * CHE, CHE)]
        iids = iid_v.at[pl.ds(j * CHE, CHE)]
        cps = []
        for k in range(EMB_DIM):
            cps.append(pltpu.make_async_copy(
                ucol_hbm[k].at[uids], ue_v.at[p, k], sem.at[p]))
            cps.append(pltpu.make_async_copy(
                icol_hbm[k].at[iids], ie_v.at[p, k], sem.at[p]))
        cps.append(pltpu.make_async_copy(ubias_hbm.at[uids], ub_v.at[p],
                                         sem.at[p]))
        cps.append(pltpu.make_async_copy(ibias_hbm.at[iids], ib_v.at[p],
                                         sem.at[p]))
        return cps

    def issue_chunk(j, p):
        for c in _chunk_cps(j, p):
            c.start()

    issue_chunk(0, 0)

    def chunk_body(j, carry):
        p = lax.rem(j, 2)
        q = 1 - p

        @pl.when(j < NCHE - 1)
        def _issue_next():
            issue_chunk(j + 1, q)

        for c in _chunk_cps(j, p):
            c.wait()

        for gg in range(CHE // L):
            ds = pl.ds(gg * L, L)
            acc = ue_v[p, 0, ds] * ie_v[p, 0, ds]
            for k in range(1, EMB_DIM):
                acc = acc + ue_v[p, k, ds] * ie_v[p, k, ds]
            acc = acc + ub_v[p, ds] + ib_v[p, ds] + bias_vec
            out_v[pl.ds(j * CHE + gg * L, L)] = acc
        return carry

    lax.fori_loop(0, NCHE, chunk_body, 0)
    pltpu.sync_copy(out_v, out_hbm.at[pl.ds(base, BPW)])


def kernel(user_ids, item_ids, user_emb_w, item_emb_w, user_text_w,
           item_text_w, user_bias, item_bias, bias):
    uidf = lax.bitcast_convert_type(user_ids, jnp.float32).reshape(NW, BPW)
    iidf = lax.bitcast_convert_type(item_ids, jnp.float32).reshape(NW, BPW)
    ucols = [user_emb_w[:, k] for k in range(EMB_DIM)]
    icols = [item_emb_w[:, k] for k in range(EMB_DIM)]
    bias16 = jnp.broadcast_to(bias, (L,))
    out_t = _sc_text(uidf, iidf, user_text_w, item_text_w)
    out_e = _sc_emb(uidf, iidf, *ucols, *icols, user_bias, item_bias,
                    bias16)
    return (out_t + out_e)[:, None]


# speedup vs baseline: 1.7501x; 1.7501x over previous
import functools

import jax
import jax.numpy as jnp
from jax import lax
from jax.experimental import pallas as pl
from jax.experimental.pallas import tpu as pltpu
from jax.experimental.pallas import tpu_sc as plsc

B = 16384
EMB_DIM = 32
BERT_DIM = 768
L = 16
NC, NS = 2, 16
NW = NC * NS
BPW = B // NW
EPR = 128 // EMB_DIM
N_EMB_ROWS = 100000 * EMB_DIM // 128

CHT = 32
NCHT = BPW // CHT
CHE = 64
NCHE = BPW // CHE

_GATHER_DNUMS = lax.GatherDimensionNumbers(
    offset_dims=(), collapsed_slice_dims=(0,), start_index_map=(0,))


def _lane_shuffle(v, idx):
    return lax.gather(v, idx[:, None], _GATHER_DNUMS, (1,),
                      mode=lax.GatherScatterMode.PROMISE_IN_BOUNDS)


def _mesh():
    return plsc.VectorSubcoreMesh(core_axis_name="c", subcore_axis_name="s")


def _load_ids(uidf_hbm, iidf_hbm, uidf_v, iidf_v, idq, wid, sem):
    idq[pl.ds(0, L)] = jnp.broadcast_to(wid, (L,)).astype(jnp.int32)
    row = idq.at[pl.ds(0, 1)]
    cu = pltpu.make_async_copy(uidf_hbm.at[row], uidf_v, sem)
    ci = pltpu.make_async_copy(iidf_hbm.at[row], iidf_v, sem)
    cu.start()
    ci.start()
    cu.wait()
    ci.wait()


def _store_i32(dst_ref, ds, value):
    dst_ref[ds] = value


@functools.partial(
    pl.kernel,
    out_type=jax.ShapeDtypeStruct((B,), jnp.float32),
    mesh=_mesh(),
    compiler_params=pltpu.CompilerParams(needs_layout_passes=False),
    scratch_types=[
        pltpu.VMEM((1, BPW), jnp.float32),
        pltpu.VMEM((1, BPW), jnp.float32),
        pltpu.VMEM((L,), jnp.int32),
        pltpu.VMEM((2, CHT), jnp.int32),
        pltpu.VMEM((2, CHT), jnp.int32),
        pltpu.VMEM((2, CHT, BERT_DIM), jnp.float32),
        pltpu.VMEM((2, CHT, BERT_DIM), jnp.float32),
        pltpu.VMEM((BPW,), jnp.float32),
        pltpu.SemaphoreType.DMA((2,)),
    ],
)
def _sc_text(uidf_hbm, iidf_hbm, utext_hbm, itext_hbm, out_hbm,
             uidf_v, iidf_v, idq, uix_c, iix_c, ut_v, it_v, out_v, sem):
    wid = lax.axis_index("s") * NC + lax.axis_index("c")
    base = wid * BPW
    _load_ids(uidf_hbm, iidf_hbm, uidf_v, iidf_v, idq, wid, sem.at[0])
    lane = lax.iota(jnp.int32, L)
    lane0 = lane == 0

    def issue_chunk(j, p):
        for g in range(CHT // L):
            ds = pl.ds(g * L, L)
            uix_c[p, ds] = plsc.bitcast(
                uidf_v[0, pl.ds(j * CHT + g * L, L)], jnp.int32)
            iix_c[p, ds] = plsc.bitcast(
                iidf_v[0, pl.ds(j * CHT + g * L, L)], jnp.int32)
        cps = _chunk_cps(p)
        for c in cps:
            c.start()

    def _chunk_cps(p):
        return [
            pltpu.make_async_copy(utext_hbm.at[uix_c.at[p]], ut_v.at[p],
                                  sem.at[p]),
            pltpu.make_async_copy(itext_hbm.at[iix_c.at[p]], it_v.at[p],
                                  sem.at[p]),
        ]

    issue_chunk(0, 0)

    def chunk_body(j, carry):
        p = lax.rem(j, 2)
        q = 1 - p

        @pl.when(j < NCHT - 1)
        def _issue_next():
            issue_chunk(j + 1, q)

        for c in _chunk_cps(p):
            c.wait()

        def pair_body(i, carry2):
            acc = ut_v[p, i, pl.ds(0, L)] * it_v[p, i, pl.ds(0, L)]
            for t in range(1, BERT_DIM // L):
                acc = acc + (ut_v[p, i, pl.ds(t * L, L)]
                             * it_v[p, i, pl.ds(t * L, L)])
            for sh in (8, 4, 2, 1):
                acc = acc + _lane_shuffle(acc, lane ^ sh)
            pos = jnp.broadcast_to(j * CHT + i, (L,)).astype(jnp.int32)
            plsc.store_scatter(out_v, [pos], acc, mask=lane0)
            return carry2

        lax.fori_loop(0, CHT, pair_body, 0)
        return carry

    lax.fori_loop(0, NCHT, chunk_body, 0)
    pltpu.sync_copy(out_v, out_hbm.at[pl.ds(base, BPW)])


@functools.partial(
    pl.kernel,
    out_type=jax.ShapeDtypeStruct((B,), jnp.float32),
    mesh=_mesh(),
    compiler_params=pltpu.CompilerParams(needs_layout_passes=False),
    scratch_types=[
        pltpu.VMEM((1, BPW), jnp.float32),
        pltpu.VMEM((1, BPW), jnp.float32),
        pltpu.VMEM((L,), jnp.int32),
        pltpu.VMEM((BPW,), jnp.int32),
        pltpu.VMEM((BPW,), jnp.int32),
        pltpu.VMEM((2, CHE), jnp.int32),
        pltpu.VMEM((2, CHE), jnp.int32),
        pltpu.VMEM((2, CHE, 128), jnp.float32),
        pltpu.VMEM((2, CHE, 128), jnp.float32),
        pltpu.VMEM((2, CHE), jnp.float32),
        pltpu.VMEM((2, CHE), jnp.float32),
        pltpu.VMEM((BPW,), jnp.float32),
        pltpu.VMEM((L,), jnp.float32),
        pltpu.SemaphoreType.DMA((2,)),
    ],
)
def _sc_emb(uidf_hbm, iidf_hbm, uemb_hbm, iemb_hbm, ubias_hbm, ibias_hbm,
            bias16_hbm, out_hbm,
            uidf_v, iidf_v, idq, uid_v, iid_v, urow_c, irow_c, ue_v, ie_v,
            ub_v, ib_v, out_v, bias_v, sem):
    wid = lax.axis_index("s") * NC + lax.axis_index("c")
    base = wid * BPW
    _load_ids(uidf_hbm, iidf_hbm, uidf_v, iidf_v, idq, wid, sem.at[0])
    pltpu.sync_copy(bias16_hbm, bias_v)
    bias_vec = bias_v[pl.ds(0, L)]

    def ids_body(g, carry):
        ds = pl.ds(g * L, L)
        uid_v[ds] = plsc.bitcast(uidf_v[0, ds], jnp.int32)
        iid_v[ds] = plsc.bitcast(iidf_v[0, ds], jnp.int32)
        return carry

    lax.fori_loop(0, BPW // L, ids_body, 0)

    lane = lax.iota(jnp.int32, L)
    lane0 = lane == 0

    def _chunk_cps(j, p):
        uids = uid_v.at[pl.ds(j * CHE, CHE)]
        iids = iid_v.at[pl.ds(j * CHE, CHE)]
        return [
            pltpu.make_async_copy(uemb_hbm.at[urow_c.at[p]], ue_v.at[p],
                                  sem.at[p]),
            pltpu.make_async_copy(iemb_hbm.at[irow_c.at[p]], ie_v.at[p],
                                  sem.at[p]),
            pltpu.make_async_copy(ubias_hbm.at[uids], ub_v.at[p], sem.at[p]),
            pltpu.make_async_copy(ibias_hbm.at[iids], ib_v.at[p], sem.at[p]),
        ]

    def issue_chunk(j, p):
        for g in range(CHE // L):
            ds = pl.ds(g * L, L)
            urow_c[p, ds] = lax.shift_right_logical(
                uid_v[pl.ds(j * CHE + g * L, L)], 2)
            irow_c[p, ds] = lax.shift_right_logical(
                iid_v[pl.ds(j * CHE + g * L, L)], 2)
        for c in _chunk_cps(j, p):
            c.start()

    issue_chunk(0, 0)

    def chunk_body(j, carry):
        p = lax.rem(j, 2)
        q = 1 - p

        @pl.when(j < NCHE - 1)
        def _issue_next():
            issue_chunk(j + 1, q)

        for c in _chunk_cps(j, p):
            c.wait()

        def pair_body(i, carry2):
            grp = j * CHE + i - lax.rem(i, L)
            pos_in_grp = jnp.broadcast_to(lax.rem(i, L), (L,))
            uid_b = _lane_shuffle(uid_v[pl.ds(grp, L)], pos_in_grp)
            iid_b = _lane_shuffle(iid_v[pl.ds(grp, L)], pos_in_grp)
            uq = (uid_b & (EPR - 1)) * EMB_DIM + lane
            iq = (iid_b & (EPR - 1)) * EMB_DIM + lane
            i_b = jnp.broadcast_to(i, (L,)).astype(jnp.int32)
            p_b = jnp.broadcast_to(p, (L,)).astype(jnp.int32)
            ue0 = plsc.load_gather(ue_v, [p_b, i_b, uq])
            ie0 = plsc.load_gather(ie_v, [p_b, i_b, iq])
            ue1 = plsc.load_gather(ue_v, [p_b, i_b, uq + L])
            ie1 = plsc.load_gather(ie_v, [p_b, i_b, iq + L])
            acc = ue0 * ie0 + ue1 * ie1
            for sh in (8, 4, 2, 1):
                acc = acc + _lane_shuffle(acc, lane ^ sh)
            pos = jnp.broadcast_to(j * CHE + i, (L,)).astype(jnp.int32)
            plsc.store_scatter(out_v, [pos], acc, mask=lane0)
            return carry2

        lax.fori_loop(0, CHE, pair_body, 0)

        for gg in range(CHE // L):
            off = j * CHE + gg * L
            out_v[pl.ds(off, L)] = (out_v[pl.ds(off, L)]
                                    + ub_v[p, pl.ds(gg * L, L)]
                                    + ib_v[p, pl.ds(gg * L, L)] + bias_vec)
        return carry

    lax.fori_loop(0, NCHE, chunk_body, 0)
    pltpu.sync_copy(out_v, out_hbm.at[pl.ds(base, BPW)])


def kernel(user_ids, item_ids, user_emb_w, item_emb_w, user_text_w,
           item_text_w, user_bias, item_bias, bias):
    uidf = lax.bitcast_convert_type(user_ids, jnp.float32).reshape(NW, BPW)
    iidf = lax.bitcast_convert_type(item_ids, jnp.float32).reshape(NW, BPW)
    uemb2 = user_emb_w.reshape(N_EMB_ROWS, 128)
    iemb2 = item_emb_w.reshape(N_EMB_ROWS, 128)
    bias16 = jnp.broadcast_to(bias, (L,))
    out_t = _sc_text(uidf, iidf, user_text_w, item_text_w)
    out_e = _sc_emb(uidf, iidf, uemb2, iemb2, user_bias, item_bias, bias16)
    return (out_t + out_e)[:, None]
